# baseline (speedup 1.0000x reference)
import jax
import jax.numpy as jnp
from jax import lax
from jax.experimental import pallas as pl
from jax.experimental.pallas import tpu as pltpu
from jax.experimental.pallas import tpu_sc as plsc

_B = 4096
_T = 200
_D = 32
_V = 1000000
_N = _B * _T
_NC = 2
_NS = 16
_NW = _NC * _NS
_LANES = 16
_TO = 8
_NU = _T // _TO
_ROWS = _TO * 128
_R = _D // 8


def _body(xt_hbm, tab_hbm, pos_hbm, out_hbm,
          xv, rows_v, slab_v, pos_v, sg, swb):
    wid = lax.axis_index("s") * _NC + lax.axis_index("c")
    c = wid

    pltpu.sync_copy(pos_hbm, pos_v)

    def fire_unit(tt, pu):
        pltpu.sync_copy(
            xt_hbm.at[pl.ds(tt * _TO, _TO), pl.ds(c * 128, 128)], xv.at[pu])
        for ti in range(_TO):
            pltpu.async_copy(
                tab_hbm.at[xv.at[pu, ti]],
                rows_v.at[pu, pl.ds(ti * 128, 128)],
                sg.at[pu])

    fire_unit(0, 0)

    def unit(tt, carry):
        pu = lax.bitwise_and(tt, 1)
        pn = 1 - pu

        @pl.when(tt + 1 < _NU)
        def _():
            fire_unit(tt + 1, pn)

        def tbody(ti, carry2):
            pltpu.make_async_copy(
                tab_hbm.at[xv.at[pu, ti]],
                rows_v.at[pu, pl.ds(ti * 128, 128)],
                sg.at[pu]).wait()
            t = tt * _TO + ti
            par = lax.bitwise_and(ti, 3)
            @pl.when(t >= 4)
            def _():
                for r in range(_R):
                    pltpu.make_async_copy(
                        slab_v.at[par, r],
                        out_hbm.at[t, r, c],
                        swb).wait()
            bt = jnp.broadcast_to(t, (_LANES,))
            lrows = [
                ti * 128 + lg * _LANES + lax.iota(jnp.int32, _LANES)
                for lg in range(8)
            ]
            for r in range(_R):
                for s in range(8):
                    d = 8 * r + s
                    bc = jnp.full((_LANES,), d, jnp.int32)
                    pv = plsc.load_gather(pos_v, [bt, bc])
                    vs = [plsc.load_gather(rows_v.at[pu], [lrows[lg], bc])
                          for lg in range(8)]
                    for lg in range(8):
                        slab_v[par, r, s, pl.ds(lg * _LANES, _LANES)] = (
                            vs[lg] + pv)
            for r in range(_R):
                pltpu.async_copy(slab_v.at[par, r], out_hbm.at[t, r, c], swb)
            return carry2

        lax.fori_loop(0, _TO, tbody, 0)
        return carry

    lax.fori_loop(0, _NU, unit, 0)
    for toff in (4, 3, 2, 1):
        t = _T - toff
        par = t & 3
        for r in range(_R):
            pltpu.make_async_copy(
                slab_v.at[par, r], out_hbm.at[t, r, c], swb).wait()


def kernel(x, token_table, pos_table):
    xt = x.astype(jnp.int32).T
    mesh = plsc.VectorSubcoreMesh(core_axis_name="c", subcore_axis_name="s")
    l5 = pl.kernel(
        _body,
        out_type=jax.ShapeDtypeStruct((_T, _R, _NW, 8, 128), jnp.float32),
        mesh=mesh,
        compiler_params=pltpu.CompilerParams(
            use_tc_tiling_on_sc=False, needs_layout_passes=False),
        scratch_types=[
            pltpu.VMEM((2, _TO, 128), jnp.int32),
            pltpu.VMEM((2, _ROWS, _D), jnp.float32),
            pltpu.VMEM((4, _R, 8, 128), jnp.float32),
            pltpu.VMEM((_T, _D), jnp.float32),
            pltpu.SemaphoreType.DMA((2,)),
            pltpu.SemaphoreType.DMA,
        ],
    )(xt, token_table, pos_table)
    return l5.transpose((2, 4, 0, 1, 3)).reshape(_B, _T, _D)


# speedup vs baseline: 1.0157x; 1.0157x over previous
import jax
import jax.numpy as jnp
from jax import lax
from jax.experimental import pallas as pl
from jax.experimental.pallas import tpu as pltpu
from jax.experimental.pallas import tpu_sc as plsc

_B = 4096
_T = 200
_D = 32
_N = _B * _T
_NC = 2
_NS = 16
_NW = _NC * _NS
_RPW = _N // _NW
_SEQ_PER_CHUNK = 4
_CHUNK = _SEQ_PER_CHUNK * _T
_NCHUNK = _RPW // _CHUNK
_G = 100
_NG = _CHUNK // _G
_LANES = 16


def _body(x_hbm, tab_hbm, pos_hbm, out_hbm,
          idx_a, idx_b, rows_a, rows_b, pos_v,
          sg_a, sg_b, swb_a, swb_b):
    wid = lax.axis_index("s") * _NC + lax.axis_index("c")
    base = wid * _RPW

    pltpu.sync_copy(pos_hbm, pos_v)

    idx_bufs = (idx_a, idx_b)
    rows_bufs = (rows_a, rows_b)
    sg = (sg_a, sg_b)
    swb = (swb_a, swb_b)

    def load_chunk(c, p):
        r0 = wid * (_RPW // _G) + c * _NG
        pltpu.sync_copy(x_hbm.at[pl.ds(r0, _NG)], idx_bufs[p])
        descs = []
        for g in range(_NG):
            descs.append(pltpu.async_copy(
                tab_hbm.at[idx_bufs[p].at[g]],
                rows_bufs[p].at[pl.ds(g * _G, _G)],
                sg[p]))
        return descs

    def add_pos(p):
        rv = rows_bufs[p]

        def tbody(t, carry):
            for half in range(0, _D, _LANES):
                pv = pos_v[t, pl.ds(half, _LANES)]
                for s in range(_SEQ_PER_CHUNK):
                    plsc.addupdate(rv.at[s * _T + t, pl.ds(half, _LANES)], pv)
            return carry

        lax.fori_loop(0, _T, tbody, 0)

    wb_descs = [None, None]
    g_descs = [None, None]
    g_descs[0] = load_chunk(0, 0)
    for c in range(_NCHUNK):
        p = c % 2
        q = 1 - p
        if c + 1 < _NCHUNK:
            if wb_descs[q] is not None:
                wb_descs[q].wait()
            g_descs[q] = load_chunk(c + 1, q)
        for dsc in g_descs[p]:
            dsc.wait()
        add_pos(p)
        wb_descs[p] = pltpu.async_copy(
            rows_bufs[p], out_hbm.at[pl.ds(base + c * _CHUNK, _CHUNK)], swb[p])
    wb_descs[0].wait()
    wb_descs[1].wait()


def kernel(x, token_table, pos_table):
    x2 = x.astype(jnp.int32).reshape(_N // _G, _G)
    mesh = plsc.VectorSubcoreMesh(core_axis_name="c", subcore_axis_name="s")
    out = pl.kernel(
        _body,
        out_type=jax.ShapeDtypeStruct((_N, _D), jnp.float32),
        mesh=mesh,
        compiler_params=pltpu.CompilerParams(use_tc_tiling_on_sc=False),
        scratch_types=[
            pltpu.VMEM((_NG, _G), jnp.int32),
            pltpu.VMEM((_NG, _G), jnp.int32),
            pltpu.VMEM((_CHUNK, _D), jnp.float32),
            pltpu.VMEM((_CHUNK, _D), jnp.float32),
            pltpu.VMEM((_T, _D), jnp.float32),
            pltpu.SemaphoreType.DMA,
            pltpu.SemaphoreType.DMA,
            pltpu.SemaphoreType.DMA,
            pltpu.SemaphoreType.DMA,
        ],
    )(x2, token_table, pos_table)
    return out.reshape(_B, _T, _D)
